# Initial kernel scaffold; baseline (speedup 1.0000x reference)
import jax
import jax.numpy as jnp
from jax.experimental import pallas as pl


def kernel(x, table, W, b):
    raise NotImplementedError("write your pallas kernel here")


# speedup vs baseline: 2.2156x; 2.2156x over previous
import functools

import jax
import jax.numpy as jnp
from jax import lax
from jax.experimental import pallas as pl
from jax.experimental.pallas import tpu as pltpu
from jax.experimental.pallas import tpu_sc as plsc

_BATCH = 4096
_HIST = 200
_EMBED = 32
_OUT = 16
_NC = 2
_NS = 16
_NW = _NC * _NS
_RPW = _BATCH // _NW
_HA = 128
_HB = _HIST - _HA


def _make_pool_kernel():
    mesh = plsc.VectorSubcoreMesh(core_axis_name="c", subcore_axis_name="s")

    @functools.partial(
        pl.kernel,
        mesh=mesh,
        compiler_params=pltpu.CompilerParams(use_tc_tiling_on_sc=False),
        out_type=jax.ShapeDtypeStruct((_BATCH * _EMBED,), jnp.float32),
        scratch_types=[
            pltpu.VMEM((_RPW, _HA), jnp.int32),
            pltpu.VMEM((_RPW, _HB), jnp.int32),
            pltpu.VMEM((_HA, _EMBED), jnp.float32),
            pltpu.VMEM((_HA, _EMBED), jnp.float32),
            pltpu.VMEM((_HB, _EMBED), jnp.float32),
            pltpu.VMEM((_HB, _EMBED), jnp.float32),
            pltpu.VMEM((_RPW * _EMBED,), jnp.float32),
            pltpu.SemaphoreType.DMA,
            pltpu.SemaphoreType.DMA,
            pltpu.SemaphoreType.DMA,
            pltpu.SemaphoreType.DMA,
        ],
    )
    def pool(xa, xb, table, out, idxa_v, idxb_v, bufA0, bufA1, bufB0,
             bufB1, out_v, semA0, semA1, semB0, semB1):
        wid = lax.axis_index("s") * _NC + lax.axis_index("c")

        pltpu.sync_copy(xa.at[wid], idxa_v)
        pltpu.sync_copy(xb.at[wid], idxb_v)

        def fire(r, bufA, bufB, semA, semB):
            pltpu.async_copy(table.at[idxa_v.at[r]], bufA, semA)
            pltpu.async_copy(table.at[idxb_v.at[r]], bufB, semB)

        def drain(bufA, bufB, semA, semB):
            pltpu.make_async_copy(table.at[idxa_v.at[0]], bufA, semA).wait()
            pltpu.make_async_copy(table.at[idxb_v.at[0]], bufB, semB).wait()

        def accum(r, bufA, bufB):
            z = jnp.zeros((16,), jnp.float32)
            p = [z, z, z, z]
            q = [z, z, z, z]
            for j in range(_HA):
                p[j % 4] = p[j % 4] + bufA[j, 0:16]
                q[j % 4] = q[j % 4] + bufA[j, 16:32]
            for j in range(_HB):
                p[j % 4] = p[j % 4] + bufB[j, 0:16]
                q[j % 4] = q[j % 4] + bufB[j, 16:32]
            s0 = (p[0] + p[1]) + (p[2] + p[3])
            s1 = (q[0] + q[1]) + (q[2] + q[3])
            out_v[pl.ds(r * _EMBED, 16)] = s0
            out_v[pl.ds(r * _EMBED + 16, 16)] = s1

        fire(0, bufA0, bufB0, semA0, semB0)

        def body(i, carry):
            r0 = 2 * i
            fire(r0 + 1, bufA1, bufB1, semA1, semB1)
            drain(bufA0, bufB0, semA0, semB0)
            accum(r0, bufA0, bufB0)

            @pl.when(i < _RPW // 2 - 1)
            def _():
                fire(r0 + 2, bufA0, bufB0, semA0, semB0)

            drain(bufA1, bufB1, semA1, semB1)
            accum(r0 + 1, bufA1, bufB1)
            return carry

        lax.fori_loop(0, _RPW // 2, body, 0)
        pltpu.sync_copy(out_v, out.at[pl.ds(wid * _RPW * _EMBED,
                                            _RPW * _EMBED)])

    return pool


_pool_kernel = _make_pool_kernel()


def _mm_body(p_ref, w_ref, b_ref, o_ref):
    o_ref[...] = (
        jnp.dot(p_ref[...], w_ref[...], preferred_element_type=jnp.float32)
        + b_ref[...]
    )


def kernel(x, table, W, b):
    xi = x.astype(jnp.int32)
    xa = xi[:, :_HA].reshape(_NW, _RPW, _HA)
    xb = xi[:, _HA:].reshape(_NW, _RPW, _HB)
    pooled = _pool_kernel(xa, xb, table).reshape(_BATCH, _EMBED)
    wt = (W.T / float(_HIST)).astype(jnp.float32)
    out = pl.pallas_call(
        _mm_body,
        out_shape=jax.ShapeDtypeStruct((_BATCH, _OUT), jnp.float32),
    )(pooled, wt, b.reshape(1, _OUT))
    return out


# speedup vs baseline: 2.2157x; 1.0000x over previous
import functools

import jax
import jax.numpy as jnp
from jax import lax
from jax.experimental import pallas as pl
from jax.experimental.pallas import tpu as pltpu
from jax.experimental.pallas import tpu_sc as plsc

_BATCH = 4096
_HIST = 200
_EMBED = 32
_OUT = 16
_NC = 2
_NS = 16
_NW = _NC * _NS
_RPW = _BATCH // _NW
_HA = 128
_HB = _HIST - _HA


def _make_pool_kernel():
    mesh = plsc.VectorSubcoreMesh(core_axis_name="c", subcore_axis_name="s")

    @functools.partial(
        pl.kernel,
        mesh=mesh,
        compiler_params=pltpu.CompilerParams(use_tc_tiling_on_sc=False),
        out_type=jax.ShapeDtypeStruct((_BATCH * _EMBED,), jnp.float32),
        scratch_types=[
            pltpu.VMEM((_RPW, _HA), jnp.int32),
            pltpu.VMEM((_RPW, _HB), jnp.int32),
            pltpu.VMEM((_HA, _EMBED), jnp.float32),
            pltpu.VMEM((_HA, _EMBED), jnp.float32),
            pltpu.VMEM((_HB, _EMBED), jnp.float32),
            pltpu.VMEM((_HB, _EMBED), jnp.float32),
            pltpu.VMEM((_RPW * _EMBED,), jnp.float32),
            pltpu.SemaphoreType.DMA,
            pltpu.SemaphoreType.DMA,
            pltpu.SemaphoreType.DMA,
            pltpu.SemaphoreType.DMA,
        ],
    )
    def pool(x, table, out, idxa_v, idxb_v, bufA0, bufA1, bufB0,
             bufB1, out_v, semA0, semA1, semB0, semB1):
        wid = lax.axis_index("s") * _NC + lax.axis_index("c")

        rows = pl.ds(wid * _RPW, _RPW)
        pltpu.sync_copy(x.at[rows, pl.ds(0, _HA)], idxa_v)
        pltpu.sync_copy(x.at[rows, pl.ds(_HA, _HB)], idxb_v)

        def fire(r, bufA, bufB, semA, semB):
            pltpu.async_copy(table.at[idxa_v.at[r]], bufA, semA)
            pltpu.async_copy(table.at[idxb_v.at[r]], bufB, semB)

        def drain(bufA, bufB, semA, semB):
            pltpu.make_async_copy(table.at[idxa_v.at[0]], bufA, semA).wait()
            pltpu.make_async_copy(table.at[idxb_v.at[0]], bufB, semB).wait()

        def accum(r, bufA, bufB):
            z = jnp.zeros((16,), jnp.float32)
            p = [z, z, z, z]
            q = [z, z, z, z]
            for j in range(_HA):
                p[j % 4] = p[j % 4] + bufA[j, 0:16]
                q[j % 4] = q[j % 4] + bufA[j, 16:32]
            for j in range(_HB):
                p[j % 4] = p[j % 4] + bufB[j, 0:16]
                q[j % 4] = q[j % 4] + bufB[j, 16:32]
            s0 = (p[0] + p[1]) + (p[2] + p[3])
            s1 = (q[0] + q[1]) + (q[2] + q[3])
            out_v[pl.ds(r * _EMBED, 16)] = s0
            out_v[pl.ds(r * _EMBED + 16, 16)] = s1

        fire(0, bufA0, bufB0, semA0, semB0)

        def body(i, carry):
            r0 = 2 * i
            fire(r0 + 1, bufA1, bufB1, semA1, semB1)
            drain(bufA0, bufB0, semA0, semB0)
            accum(r0, bufA0, bufB0)

            @pl.when(i < _RPW // 2 - 1)
            def _():
                fire(r0 + 2, bufA0, bufB0, semA0, semB0)

            drain(bufA1, bufB1, semA1, semB1)
            accum(r0 + 1, bufA1, bufB1)
            return carry

        lax.fori_loop(0, _RPW // 2, body, 0)
        pltpu.sync_copy(out_v, out.at[pl.ds(wid * _RPW * _EMBED,
                                            _RPW * _EMBED)])

    return pool


_pool_kernel = _make_pool_kernel()


def _mm_body(p_ref, w_ref, b_ref, o_ref):
    o_ref[...] = (
        jnp.dot(p_ref[...], w_ref[...], preferred_element_type=jnp.float32)
        + b_ref[...]
    )


def kernel(x, table, W, b):
    xi = x.astype(jnp.int32)
    pooled = _pool_kernel(xi, table).reshape(_BATCH, _EMBED)
    wt = (W.T / float(_HIST)).astype(jnp.float32)
    out = pl.pallas_call(
        _mm_body,
        out_shape=jax.ShapeDtypeStruct((_BATCH, _OUT), jnp.float32),
    )(pooled, wt, b.reshape(1, _OUT))
    return out
